# baseline (speedup 1.0000x reference)
import functools

import jax
import jax.numpy as jnp
from jax import lax
from jax.experimental import pallas as pl
from jax.experimental.pallas import tpu as pltpu
from jax.experimental.pallas import tpu_sc as plsc

B, N, D, P = 16, 4096, 128, 2048

NC, NS, L = 2, 16, 16
NW = NC * NS

E_ROWS = 3 * B * P * 2
ROWS_PER_W = E_ROWS // NW
CHUNK = 128
NCHUNK = ROWS_PER_W // CHUNK
PAIRS_PER_W = ROWS_PER_W // 2
BP = B * P
B0_PER_W = BP // NW
B0_CHUNKS = B0_PER_W // CHUNK

_mesh = plsc.VectorSubcoreMesh(core_axis_name="c", subcore_axis_name="s")


@functools.partial(
    pl.kernel,
    out_type=[
        jax.ShapeDtypeStruct((E_ROWS // 2,), jnp.float32),
        jax.ShapeDtypeStruct((BP,), jnp.float32),
    ],
    mesh=_mesh,
    compiler_params=pltpu.CompilerParams(needs_layout_passes=False,
                                         use_tc_tiling_on_sc=False),
    scratch_types=[
        pltpu.VMEM((NCHUNK, CHUNK), jnp.int32),
        pltpu.VMEM((CHUNK, D // 2), jnp.float32),
        pltpu.VMEM((CHUNK, D // 2), jnp.float32),
        pltpu.VMEM((CHUNK, D // 2), jnp.float32),
        pltpu.VMEM((CHUNK, D // 2), jnp.float32),
        pltpu.VMEM((CHUNK,), jnp.float32),
        pltpu.VMEM((CHUNK,), jnp.float32),
        pltpu.VMEM((CHUNK,), jnp.float32),
        pltpu.VMEM((CHUNK,), jnp.float32),
        pltpu.VMEM((PAIRS_PER_W,), jnp.float32),
        pltpu.VMEM(((CHUNK // 2) * 17,), jnp.float32),
        pltpu.VMEM((B0_CHUNKS, CHUNK), jnp.int32),
        pltpu.VMEM((B0_PER_W,), jnp.float32),
        pltpu.SemaphoreType.DMA,
        pltpu.SemaphoreType.DMA,
        pltpu.SemaphoreType.DMA,
        pltpu.SemaphoreType.DMA,
        pltpu.SemaphoreType.DMA,
    ],
)
def _sc_kernel(pts_hbm, vf_hbm, eidx_hbm, bidx_hbm, outd_hbm, outb_hbm,
               idx_v, rows0_v, rows1_v, rows2_v, rows3_v,
               vfv0_v, vfv1_v, vfv2_v, vfv3_v, res_v, sums_v,
               bidx_v, b0_v, sem0, sem1, sem2, sem3, semb):
    wid = lax.axis_index("s") * NC + lax.axis_index("c")
    iota = lax.iota(jnp.int32, L)
    rows_bufs = (rows0_v, rows1_v, rows2_v, rows3_v)
    vfv_bufs = (vfv0_v, vfv1_v, vfv2_v, vfv3_v)
    sems = (sem0, sem1, sem2, sem3)
    NBUF = 4

    pltpu.sync_copy(eidx_hbm.at[pl.ds(wid * NCHUNK, NCHUNK)], idx_v)
    pltpu.sync_copy(bidx_hbm.at[pl.ds(wid * B0_CHUNKS, B0_CHUNKS)], bidx_v)

    def fire(j, b):
        pltpu.async_copy(pts_hbm.at[idx_v.at[j]], rows_bufs[b], sems[b])
        pltpu.async_copy(vf_hbm.at[idx_v.at[j]], vfv_bufs[b], sems[b])

    def wait(j, b):
        pltpu.make_async_copy(pts_hbm.at[idx_v.at[j]], rows_bufs[b],
                              sems[b]).wait()
        pltpu.make_async_copy(vf_hbm.at[idx_v.at[j]], vfv_bufs[b],
                              sems[b]).wait()

    def compute(j, b):
        rows_b = rows_bufs[b]
        vfv_b = vfv_bufs[b]

        def pair(p, carry):
            r_a = 2 * p
            acc = None
            for k in range(D // L):
                av = rows_b[r_a, pl.ds(k * L, L)]
                bv = rows_b[r_a + 1, pl.ds(k * L, L)]
                t = av - bv
                sq = t * t
                acc = sq if acc is None else acc + sq
            sums_v[pl.ds(p * 17, L)] = acc
            return carry

        lax.fori_loop(0, CHUNK // 2, pair, 0)

        def group(g, carry2):
            base = g * (L * 17) + iota * 17
            x = jnp.full((L,), 1e-12, jnp.float32)
            for k in range(L):
                x = x + plsc.load_gather(sums_v, [base + k])
            y = plsc.bitcast(
                (plsc.bitcast(x, jnp.int32) >> 1) + 0x1FBD1DF5, jnp.float32)
            y = 0.5 * (y + x / y)
            y = 0.5 * (y + x / y)
            y = 0.5 * (y + x / y)
            f_a = plsc.load_gather(vfv_b, [iota * 2 + g * 32])
            f_b = plsc.load_gather(vfv_b, [iota * 2 + g * 32 + 1])
            res_v[pl.ds(j * (CHUNK // 2) + g * L, L)] = y + jnp.maximum(f_a, f_b)
            return carry2

        lax.fori_loop(0, CHUNK // (2 * L), group, 0)

    for b in range(NBUF):
        fire(b, b)

    def step(i, carry):
        for b in range(NBUF):
            j = NBUF * i + b
            wait(j, b)

            @pl.when(j + NBUF < NCHUNK)
            def _():
                fire(j + NBUF, b)
        return carry

    lax.fori_loop(0, NCHUNK // NBUF, step, 0)

    def bchunk(j, carry):
        pltpu.async_copy(vf_hbm.at[bidx_v.at[j]],
                         b0_v.at[pl.ds(j * CHUNK, CHUNK)], semb).wait()
        return carry

    lax.fori_loop(0, B0_CHUNKS, bchunk, 0)

    pltpu.sync_copy(res_v, outd_hbm.at[pl.ds(wid * PAIRS_PER_W, PAIRS_PER_W)])
    pltpu.sync_copy(b0_v, outb_hbm.at[pl.ds(wid * B0_PER_W, B0_PER_W)])


def kernel(pts, vFilts, ppb0, ppd0, ppb1, ppd1):
    pts_bf = pts.astype(jnp.bfloat16).reshape(B * N, D // 2, 2)
    pts_flat = jax.lax.bitcast_convert_type(pts_bf, jnp.float32)
    vf_flat = vFilts.reshape(B * N)
    offs = (jnp.arange(B, dtype=jnp.int32) * N)[:, None, None]
    eidx = jnp.stack([
        ppd0.astype(jnp.int32) + offs,
        ppb1.astype(jnp.int32) + offs,
        ppd1.astype(jnp.int32) + offs,
    ])
    eidx = eidx.reshape(E_ROWS // CHUNK, CHUNK)
    bidx = (ppb0.astype(jnp.int32) + offs[:, :, 0]).reshape(BP // CHUNK, CHUNK)

    outd, outb = _sc_kernel(pts_flat, vf_flat, eidx, bidx)

    d3 = outd.reshape(3, B, P)
    b0 = outb.reshape(B, P)
    pd0 = jnp.stack([b0, d3[0]], axis=-1)
    pd1 = jnp.stack([d3[1], d3[2]], axis=-1)
    return jnp.stack([pd0, pd1], axis=0)


# speedup vs baseline: 2.0126x; 2.0126x over previous
import functools

import jax
import jax.numpy as jnp
from jax import lax
from jax.experimental import pallas as pl
from jax.experimental.pallas import tpu as pltpu
from jax.experimental.pallas import tpu_sc as plsc

B, N, D, P = 16, 4096, 128, 2048

NC, NS, L = 2, 16, 16
NW = NC * NS

E_ROWS = 3 * B * P * 2
ROWS_PER_W = E_ROWS // NW
CHUNK = 128
NCHUNK = ROWS_PER_W // CHUNK
PAIRS_PER_W = ROWS_PER_W // 2
BP = B * P
B0_PER_W = BP // NW
NBUF = 2

_mesh = plsc.VectorSubcoreMesh(core_axis_name="c", subcore_axis_name="s")


@functools.partial(
    pl.kernel,
    out_type=[
        jax.ShapeDtypeStruct((E_ROWS // 2,), jnp.float32),
        jax.ShapeDtypeStruct((BP,), jnp.float32),
    ],
    mesh=_mesh,
    compiler_params=pltpu.CompilerParams(needs_layout_passes=False),
    scratch_types=[
        pltpu.VMEM((B * N,), jnp.float32),
        pltpu.VMEM((NCHUNK, CHUNK), jnp.int32),
        pltpu.VMEM((CHUNK, D), jnp.float32),
        pltpu.VMEM((CHUNK, D), jnp.float32),
        pltpu.VMEM((PAIRS_PER_W,), jnp.float32),
        pltpu.VMEM(((CHUNK // 2) * 17,), jnp.float32),
        pltpu.VMEM((B0_PER_W // CHUNK, CHUNK), jnp.int32),
        pltpu.VMEM((B0_PER_W,), jnp.float32),
        pltpu.SemaphoreType.DMA,
        pltpu.SemaphoreType.DMA,
    ],
)
def _sc_kernel(pts_hbm, vf_hbm, eidx_hbm, bidx_hbm, outd_hbm, outb_hbm,
               vft_v, idx_v, rows0_v, rows1_v, res_v, sums_v,
               bidx_v, b0_v, sem0, sem1):
    wid = lax.axis_index("s") * NC + lax.axis_index("c")
    iota = lax.iota(jnp.int32, L)
    rows_bufs = (rows0_v, rows1_v)
    sems = (sem0, sem1)

    pltpu.sync_copy(eidx_hbm.at[pl.ds(wid * NCHUNK, NCHUNK)], idx_v)
    pltpu.sync_copy(bidx_hbm.at[pl.ds(wid * (B0_PER_W // CHUNK),
                                      B0_PER_W // CHUNK)], bidx_v)
    pltpu.sync_copy(vf_hbm, vft_v)

    def fire(j, b):
        pltpu.async_copy(pts_hbm.at[idx_v.at[j]], rows_bufs[b], sems[b])

    def wait(j, b):
        pltpu.make_async_copy(pts_hbm.at[idx_v.at[j]], rows_bufs[b],
                              sems[b]).wait()

    def compute(j, b):
        rows_b = rows_bufs[b]

        def pair(p, carry):
            r_a = 2 * p
            acc = None
            for k in range(D // L):
                av = rows_b[r_a, pl.ds(k * L, L)]
                bv = rows_b[r_a + 1, pl.ds(k * L, L)]
                t = av - bv
                sq = t * t
                acc = sq if acc is None else acc + sq
            sums_v[pl.ds(p * 17, L)] = acc
            return carry

        lax.fori_loop(0, CHUNK // 2, pair, 0)

        def group(g, carry2):
            base = g * (L * 17) + iota * 17
            x = jnp.full((L,), 1e-12, jnp.float32)
            for k in range(L):
                x = x + plsc.load_gather(sums_v, [base + k])
            y = plsc.bitcast(
                (plsc.bitcast(x, jnp.int32) >> 1) + 0x1FBD1DF5, jnp.float32)
            y = 0.5 * (y + x / y)
            y = 0.5 * (y + x / y)
            y = 0.5 * (y + x / y)
            jcol = jnp.full((L,), j, jnp.int32)
            ia = plsc.load_gather(idx_v, [jcol, iota * 2 + g * 32])
            ib = plsc.load_gather(idx_v, [jcol, iota * 2 + g * 32 + 1])
            f_a = plsc.load_gather(vft_v, [ia])
            f_b = plsc.load_gather(vft_v, [ib])
            res_v[pl.ds(j * (CHUNK // 2) + g * L, L)] = y + jnp.maximum(f_a, f_b)
            return carry2

        lax.fori_loop(0, CHUNK // (2 * L), group, 0)

    for b in range(NBUF):
        fire(b, b)

    def step(i, carry):
        for b in range(NBUF):
            j = NBUF * i + b
            wait(j, b)
            compute(j, b)

            @pl.when(j + NBUF < NCHUNK)
            def _():
                fire(j + NBUF, b)
        return carry

    lax.fori_loop(0, NCHUNK // NBUF, step, 0)

    for t in range(B0_PER_W // L):
        iv = bidx_v[t // (CHUNK // L), pl.ds((t % (CHUNK // L)) * L, L)]
        b0_v[pl.ds(t * L, L)] = plsc.load_gather(vft_v, [iv])

    pltpu.sync_copy(res_v, outd_hbm.at[pl.ds(wid * PAIRS_PER_W, PAIRS_PER_W)])
    pltpu.sync_copy(b0_v, outb_hbm.at[pl.ds(wid * B0_PER_W, B0_PER_W)])


def kernel(pts, vFilts, ppb0, ppd0, ppb1, ppd1):
    pts_flat = pts.reshape(B * N, D)
    vf_flat = vFilts.reshape(B * N)
    offs = (jnp.arange(B, dtype=jnp.int32) * N)[:, None, None]
    eidx = jnp.stack([
        ppd0.astype(jnp.int32) + offs,
        ppb1.astype(jnp.int32) + offs,
        ppd1.astype(jnp.int32) + offs,
    ])
    eidx = eidx.reshape(E_ROWS // CHUNK, CHUNK)
    bidx = (ppb0.astype(jnp.int32) + offs[:, :, 0]).reshape(BP // CHUNK, CHUNK)

    outd, outb = _sc_kernel(pts_flat, vf_flat, eidx, bidx)

    d3 = outd.reshape(3, B, P)
    b0 = outb.reshape(B, P)
    pd0 = jnp.stack([b0, d3[0]], axis=-1)
    pd1 = jnp.stack([d3[1], d3[2]], axis=-1)
    return jnp.stack([pd0, pd1], axis=0)


# speedup vs baseline: 2.0671x; 1.0270x over previous
import functools

import jax
import jax.numpy as jnp
from jax import lax
from jax.experimental import pallas as pl
from jax.experimental.pallas import tpu as pltpu
from jax.experimental.pallas import tpu_sc as plsc

B, N, D, P = 16, 4096, 128, 2048

NC, NS, L = 2, 16, 16
NW = NC * NS

E_ROWS = 3 * B * P * 2
ROWS_PER_W = E_ROWS // NW
CHUNK = 128
NCHUNK = ROWS_PER_W // CHUNK
PAIRS_PER_W = ROWS_PER_W // 2
BP = B * P
B0_PER_W = BP // NW
B0_CHUNKS = B0_PER_W // CHUNK
NBUF = 6

_mesh = plsc.VectorSubcoreMesh(core_axis_name="c", subcore_axis_name="s")


@functools.partial(
    pl.kernel,
    out_type=[
        jax.ShapeDtypeStruct((E_ROWS // 2,), jnp.float32),
        jax.ShapeDtypeStruct((BP,), jnp.float32),
    ],
    mesh=_mesh,
    compiler_params=pltpu.CompilerParams(needs_layout_passes=False),
    scratch_types=(
        [pltpu.VMEM((NCHUNK, CHUNK), jnp.int32)]
        + [pltpu.VMEM((CHUNK, D), jnp.float32)] * NBUF
        + [pltpu.VMEM((CHUNK,), jnp.float32)] * NBUF
        + [
            pltpu.VMEM((PAIRS_PER_W,), jnp.float32),
            pltpu.VMEM(((CHUNK // 2) * 17,), jnp.float32),
            pltpu.VMEM((B0_CHUNKS, CHUNK), jnp.int32),
            pltpu.VMEM((B0_PER_W,), jnp.float32),
        ]
        + [pltpu.SemaphoreType.DMA] * (NBUF + 1)
    ),
)
def _sc_kernel(pts_hbm, vf_hbm, eidx_hbm, bidx_hbm, outd_hbm, outb_hbm,
               idx_v, *rest):
    rows_bufs = rest[:NBUF]
    vfv_bufs = rest[NBUF:2 * NBUF]
    res_v, sums_v, bidx_v, b0_v = rest[2 * NBUF:2 * NBUF + 4]
    sems = rest[2 * NBUF + 4:3 * NBUF + 4]
    semb = rest[3 * NBUF + 4]

    wid = lax.axis_index("s") * NC + lax.axis_index("c")
    iota = lax.iota(jnp.int32, L)

    pltpu.sync_copy(eidx_hbm.at[pl.ds(wid * NCHUNK, NCHUNK)], idx_v)
    pltpu.sync_copy(bidx_hbm.at[pl.ds(wid * B0_CHUNKS, B0_CHUNKS)], bidx_v)

    def fire(j, b):
        pltpu.async_copy(pts_hbm.at[idx_v.at[j]], rows_bufs[b], sems[b])
        pltpu.async_copy(vf_hbm.at[idx_v.at[j]], vfv_bufs[b], sems[b])

    def wait(j, b):
        pltpu.make_async_copy(pts_hbm.at[idx_v.at[j]], rows_bufs[b],
                              sems[b]).wait()
        pltpu.make_async_copy(vf_hbm.at[idx_v.at[j]], vfv_bufs[b],
                              sems[b]).wait()

    def compute(j, b):
        rows_b = rows_bufs[b]
        vfv_b = vfv_bufs[b]

        def pair(p, carry):
            r_a = 2 * p
            acc = None
            for k in range(D // L):
                av = rows_b[r_a, pl.ds(k * L, L)]
                bv = rows_b[r_a + 1, pl.ds(k * L, L)]
                t = av - bv
                sq = t * t
                acc = sq if acc is None else acc + sq
            sums_v[pl.ds(p * 17, L)] = acc
            return carry

        lax.fori_loop(0, CHUNK // 2, pair, 0)

        def group(g, carry2):
            base = g * (L * 17) + iota * 17
            x = jnp.full((L,), 1e-12, jnp.float32)
            for k in range(L):
                x = x + plsc.load_gather(sums_v, [base + k])
            y = plsc.bitcast(
                (plsc.bitcast(x, jnp.int32) >> 1) + 0x1FBD1DF5, jnp.float32)
            y = 0.5 * (y + x / y)
            y = 0.5 * (y + x / y)
            y = 0.5 * (y + x / y)
            f_a = plsc.load_gather(vfv_b, [iota * 2 + g * 32])
            f_b = plsc.load_gather(vfv_b, [iota * 2 + g * 32 + 1])
            res_v[pl.ds(j * (CHUNK // 2) + g * L, L)] = y + jnp.maximum(f_a, f_b)
            return carry2

        lax.fori_loop(0, CHUNK // (2 * L), group, 0)

    for b in range(NBUF):
        fire(b, b)

    def step(i, carry):
        for b in range(NBUF):
            j = NBUF * i + b
            wait(j, b)
            compute(j, b)

            @pl.when(j + NBUF < NCHUNK)
            def _():
                fire(j + NBUF, b)
        return carry

    lax.fori_loop(0, NCHUNK // NBUF, step, 0)

    def bchunk(j, carry):
        pltpu.async_copy(vf_hbm.at[bidx_v.at[j]],
                         b0_v.at[pl.ds(j * CHUNK, CHUNK)], semb).wait()
        return carry

    lax.fori_loop(0, B0_CHUNKS, bchunk, 0)

    pltpu.sync_copy(res_v, outd_hbm.at[pl.ds(wid * PAIRS_PER_W, PAIRS_PER_W)])
    pltpu.sync_copy(b0_v, outb_hbm.at[pl.ds(wid * B0_PER_W, B0_PER_W)])


def kernel(pts, vFilts, ppb0, ppd0, ppb1, ppd1):
    pts_flat = pts.reshape(B * N, D)
    vf_flat = vFilts.reshape(B * N)
    offs = (jnp.arange(B, dtype=jnp.int32) * N)[:, None, None]
    eidx = jnp.stack([
        ppd0.astype(jnp.int32) + offs,
        ppb1.astype(jnp.int32) + offs,
        ppd1.astype(jnp.int32) + offs,
    ])
    eidx = eidx.reshape(E_ROWS // CHUNK, CHUNK)
    bidx = (ppb0.astype(jnp.int32) + offs[:, :, 0]).reshape(BP // CHUNK, CHUNK)

    outd, outb = _sc_kernel(pts_flat, vf_flat, eidx, bidx)

    d3 = outd.reshape(3, B, P)
    b0 = outb.reshape(B, P)
    pd0 = jnp.stack([b0, d3[0]], axis=-1)
    pd1 = jnp.stack([d3[1], d3[2]], axis=-1)
    return jnp.stack([pd0, pd1], axis=0)


# speedup vs baseline: 2.1669x; 1.0483x over previous
import functools

import jax
import jax.numpy as jnp
from jax import lax
from jax.experimental import pallas as pl
from jax.experimental.pallas import tpu as pltpu
from jax.experimental.pallas import tpu_sc as plsc

B, N, D, P = 16, 4096, 128, 2048

NC, NS, L = 2, 16, 16
NW = NC * NS

E_ROWS = 3 * B * P * 2
ROWS_PER_W = E_ROWS // NW
CHUNK = 128
NCHUNK = ROWS_PER_W // CHUNK
PAIRS_PER_W = ROWS_PER_W // 2
BP = B * P
B0_PER_W = BP // NW
B0_CHUNKS = B0_PER_W // CHUNK
NBUF = 4

_mesh = plsc.VectorSubcoreMesh(core_axis_name="c", subcore_axis_name="s")


@functools.partial(
    pl.kernel,
    out_type=[
        jax.ShapeDtypeStruct((E_ROWS // 2,), jnp.float32),
        jax.ShapeDtypeStruct((BP,), jnp.float32),
    ],
    mesh=_mesh,
    compiler_params=pltpu.CompilerParams(needs_layout_passes=False),
    scratch_types=(
        [pltpu.VMEM((NCHUNK, CHUNK), jnp.int32)]
        + [pltpu.VMEM((CHUNK, D), jnp.float32)] * NBUF
        + [pltpu.VMEM((CHUNK,), jnp.float32)] * NBUF
        + [
            pltpu.VMEM((PAIRS_PER_W,), jnp.float32),
            pltpu.VMEM(((CHUNK // 2) * 17,), jnp.float32),
            pltpu.VMEM((B0_CHUNKS, CHUNK), jnp.int32),
            pltpu.VMEM((B0_PER_W,), jnp.float32),
        ]
        + [pltpu.SemaphoreType.DMA] * (NBUF + 1)
    ),
)
def _sc_kernel(pts_hbm, vf_hbm, eidx_hbm, bidx_hbm, outd_hbm, outb_hbm,
               idx_v, *rest):
    rows_bufs = rest[:NBUF]
    vfv_bufs = rest[NBUF:2 * NBUF]
    res_v, sums_v, bidx_v, b0_v = rest[2 * NBUF:2 * NBUF + 4]
    sems = rest[2 * NBUF + 4:3 * NBUF + 4]
    semb = rest[3 * NBUF + 4]

    wid = lax.axis_index("s") * NC + lax.axis_index("c")
    iota = lax.iota(jnp.int32, L)

    pltpu.sync_copy(eidx_hbm.at[pl.ds(wid * NCHUNK, NCHUNK)], idx_v)
    pltpu.sync_copy(bidx_hbm.at[pl.ds(wid * B0_CHUNKS, B0_CHUNKS)], bidx_v)

    def fire(j, b):
        pltpu.async_copy(pts_hbm.at[idx_v.at[j]], rows_bufs[b], sems[b])
        pltpu.async_copy(vf_hbm.at[idx_v.at[j]], vfv_bufs[b], sems[b])

    def wait(j, b):
        pltpu.make_async_copy(pts_hbm.at[idx_v.at[j]], rows_bufs[b],
                              sems[b]).wait()
        pltpu.make_async_copy(vf_hbm.at[idx_v.at[j]], vfv_bufs[b],
                              sems[b]).wait()

    def compute(j, b):
        rows_b = rows_bufs[b]
        vfv_b = vfv_bufs[b]

        def pair(p, carry):
            r_a = 2 * p
            acc = None
            for k in range(D // L):
                av = rows_b[r_a, pl.ds(k * L, L)]
                bv = rows_b[r_a + 1, pl.ds(k * L, L)]
                t = av - bv
                sq = t * t
                acc = sq if acc is None else acc + sq
            sums_v[pl.ds(p * 17, L)] = acc
            return carry

        lax.fori_loop(0, CHUNK // 2, pair, 0)

        def group(g, carry2):
            base = g * (L * 17) + iota * 17
            x = jnp.full((L,), 1e-12, jnp.float32)
            for k in range(L):
                x = x + plsc.load_gather(sums_v, [base + k])
            y = plsc.bitcast(
                (plsc.bitcast(x, jnp.int32) >> 1) + 0x1FBD1DF5, jnp.float32)
            y = 0.5 * (y + x / y)
            y = 0.5 * (y + x / y)
            y = 0.5 * (y + x / y)
            f_a = plsc.load_gather(vfv_b, [iota * 2 + g * 32])
            f_b = plsc.load_gather(vfv_b, [iota * 2 + g * 32 + 1])
            res_v[pl.ds(j * (CHUNK // 2) + g * L, L)] = y + jnp.maximum(f_a, f_b)
            return carry2

        lax.fori_loop(0, CHUNK // (2 * L), group, 0)

    def bfire(j, carry):
        pltpu.async_copy(vf_hbm.at[bidx_v.at[j]],
                         b0_v.at[pl.ds(j * CHUNK, CHUNK)], semb)
        return carry

    lax.fori_loop(0, B0_CHUNKS, bfire, 0)

    for b in range(NBUF):
        fire(b, b)

    def step(i, carry):
        for b in range(NBUF):
            j = NBUF * i + b
            wait(j, b)
            compute(j, b)

            @pl.when(j + NBUF < NCHUNK)
            def _():
                fire(j + NBUF, b)
        return carry

    lax.fori_loop(0, NCHUNK // NBUF, step, 0)

    def bwait(j, carry):
        pltpu.make_async_copy(vf_hbm.at[bidx_v.at[j]],
                              b0_v.at[pl.ds(j * CHUNK, CHUNK)], semb).wait()
        return carry

    lax.fori_loop(0, B0_CHUNKS, bwait, 0)

    pltpu.sync_copy(res_v, outd_hbm.at[pl.ds(wid * PAIRS_PER_W, PAIRS_PER_W)])
    pltpu.sync_copy(b0_v, outb_hbm.at[pl.ds(wid * B0_PER_W, B0_PER_W)])


def kernel(pts, vFilts, ppb0, ppd0, ppb1, ppd1):
    pts_flat = pts.reshape(B * N, D)
    vf_flat = vFilts.reshape(B * N)
    offs = (jnp.arange(B, dtype=jnp.int32) * N)[:, None, None]
    eidx = jnp.stack([
        ppd0.astype(jnp.int32) + offs,
        ppb1.astype(jnp.int32) + offs,
        ppd1.astype(jnp.int32) + offs,
    ])
    eidx = eidx.reshape(E_ROWS // CHUNK, CHUNK)
    bidx = (ppb0.astype(jnp.int32) + offs[:, :, 0]).reshape(BP // CHUNK, CHUNK)

    outd, outb = _sc_kernel(pts_flat, vf_flat, eidx, bidx)

    d3 = outd.reshape(3, B, P)
    b0 = outb.reshape(B, P)
    pd0 = jnp.stack([b0, d3[0]], axis=-1)
    pd1 = jnp.stack([d3[1], d3[2]], axis=-1)
    return jnp.stack([pd0, pd1], axis=0)
